# baseline (speedup 1.0000x reference)
import functools

import jax
import jax.numpy as jnp
from jax.experimental import pallas as pl

_BETA = 0.25
_K = 1024
_C = 256
_B = 4
_G = 16
_T = 512


def _vq_body(zs_ref, z_ref, emb_ref, embT_ref, es_ref, zq_ref, inds_ref,
             loss_ref):
    b = pl.program_id(0)
    g = pl.program_id(1)

    zb = z_ref[0, :, 0, 0, :]
    dot = jax.lax.dot_general(
        emb_ref[...], zb, (((1,), (0,)), ((), ())),
        preferred_element_type=jnp.float32)
    zs_row = zs_ref[0, 0]
    d = (es_ref[...] + zs_row) - 2.0 * dot

    m = jnp.min(d, axis=0, keepdims=True)
    iota_k = jax.lax.broadcasted_iota(jnp.int32, (_K, _T), 0)
    idx = jnp.min(jnp.where(d == m, iota_k, _K), axis=0, keepdims=True)
    inds_ref[0, 0] = idx

    onehot = (iota_k == idx).astype(jnp.float32)
    zq = jax.lax.dot_general(
        embT_ref[...], onehot, (((1,), (0,)), ((), ())),
        preferred_element_type=jnp.float32)
    zq_ref[0, :, 0, 0, :] = zb + (zq - zb)

    @pl.when(jnp.logical_and(b == 0, g == 0))
    def _init():
        loss_ref[...] = jnp.zeros_like(loss_ref)

    loss_ref[...] += jnp.sum(m, axis=(0, 1), keepdims=True).reshape(1, 1)


@functools.partial(jax.jit, static_argnames=())
def kernel(z, embedding):
    B, C, D, H, W = z.shape
    K = embedding.shape[0]
    z5 = z.reshape(B, C, _G, 1, _T)
    zp = jnp.transpose(z, (0, 2, 3, 4, 1))
    zs = jnp.sum(zp.reshape(-1, C) ** 2, axis=1).reshape(B, _G, 1, _T)
    es = jnp.sum(embedding ** 2, axis=1).reshape(K, 1)
    embT = embedding.T

    zq5, inds4, loss_acc = pl.pallas_call(
        _vq_body,
        grid=(B, _G),
        in_specs=[
            pl.BlockSpec((1, 1, 1, _T), lambda b, g: (b, g, 0, 0)),
            pl.BlockSpec((1, C, 1, 1, _T), lambda b, g: (b, 0, g, 0, 0)),
            pl.BlockSpec((K, C), lambda b, g: (0, 0)),
            pl.BlockSpec((C, K), lambda b, g: (0, 0)),
            pl.BlockSpec((K, 1), lambda b, g: (0, 0)),
        ],
        out_specs=[
            pl.BlockSpec((1, C, 1, 1, _T), lambda b, g: (b, 0, g, 0, 0)),
            pl.BlockSpec((1, 1, 1, _T), lambda b, g: (b, g, 0, 0)),
            pl.BlockSpec((1, 1), lambda b, g: (0, 0)),
        ],
        out_shape=[
            jax.ShapeDtypeStruct((B, C, _G, 1, _T), jnp.float32),
            jax.ShapeDtypeStruct((B, _G, 1, _T), jnp.int32),
            jax.ShapeDtypeStruct((1, 1), jnp.float32),
        ],
    )(zs, z5, embedding, embT, es)

    z_q_out = zq5.reshape(B, C, D, H, W)
    inds = inds4.reshape(B, D, H, W)
    loss = loss_acc[0, 0] * (_BETA / (B * D * H * W * C))
    return (z_q_out, inds, loss)


# speedup vs baseline: 1.0182x; 1.0182x over previous
import functools

import jax
import jax.numpy as jnp
from jax.experimental import pallas as pl

_BETA = 0.25
_K = 1024
_C = 256
_B = 4
_G = 16
_T = 512


def _vq_body(z_ref, emb_ref, embT_ref, es_ref, zq_ref, inds_ref,
             loss_ref):
    b = pl.program_id(0)
    g = pl.program_id(1)

    zb = z_ref[0, :, 0, 0, :]
    dot = jax.lax.dot_general(
        emb_ref[...], zb, (((1,), (0,)), ((), ())),
        preferred_element_type=jnp.float32)
    zs_row = jnp.sum(zb * zb, axis=0, keepdims=True)
    d = (es_ref[...] + zs_row) - 2.0 * dot

    m = jnp.min(d, axis=0, keepdims=True)
    iota_k = jax.lax.broadcasted_iota(jnp.int32, (_K, _T), 0)
    idx = jnp.min(jnp.where(d == m, iota_k, _K), axis=0, keepdims=True)
    inds_ref[0, 0] = idx

    onehot = (iota_k == idx).astype(jnp.float32)
    zq = jax.lax.dot_general(
        embT_ref[...], onehot, (((1,), (0,)), ((), ())),
        preferred_element_type=jnp.float32)
    zq_ref[0, :, 0, 0, :] = zb + (zq - zb)

    @pl.when(jnp.logical_and(b == 0, g == 0))
    def _init():
        loss_ref[...] = jnp.zeros_like(loss_ref)

    loss_ref[...] += jnp.sum(m, axis=(0, 1), keepdims=True).reshape(1, 1)


@functools.partial(jax.jit, static_argnames=())
def kernel(z, embedding):
    B, C, D, H, W = z.shape
    K = embedding.shape[0]
    z5 = z.reshape(B, C, _G, 1, _T)
    es = jnp.sum(embedding ** 2, axis=1).reshape(K, 1)
    embT = embedding.T

    zq5, inds4, loss_acc = pl.pallas_call(
        _vq_body,
        grid=(B, _G),
        in_specs=[
            pl.BlockSpec((1, C, 1, 1, _T), lambda b, g: (b, 0, g, 0, 0)),
            pl.BlockSpec((K, C), lambda b, g: (0, 0)),
            pl.BlockSpec((C, K), lambda b, g: (0, 0)),
            pl.BlockSpec((K, 1), lambda b, g: (0, 0)),
        ],
        out_specs=[
            pl.BlockSpec((1, C, 1, 1, _T), lambda b, g: (b, 0, g, 0, 0)),
            pl.BlockSpec((1, 1, 1, _T), lambda b, g: (b, g, 0, 0)),
            pl.BlockSpec((1, 1), lambda b, g: (0, 0)),
        ],
        out_shape=[
            jax.ShapeDtypeStruct((B, C, _G, 1, _T), jnp.float32),
            jax.ShapeDtypeStruct((B, _G, 1, _T), jnp.int32),
            jax.ShapeDtypeStruct((1, 1), jnp.float32),
        ],
    )(z5, embedding, embT, es)

    z_q_out = zq5.reshape(B, C, D, H, W)
    inds = inds4.reshape(B, D, H, W)
    loss = loss_acc[0, 0] * (_BETA / (B * D * H * W * C))
    return (z_q_out, inds, loss)
